# Initial kernel scaffold; baseline (speedup 1.0000x reference)
import jax
import jax.numpy as jnp
from jax.experimental import pallas as pl


def kernel(inviter_ids, voter_pos_ids, voter_neg_ids, h, r, pos_t, neg_t, is_train, user_embed, entity_embed, relation_embed, trans_M):
    raise NotImplementedError("write your pallas kernel here")


# speedup vs baseline: 1.5294x; 1.5294x over previous
import functools

import jax
import jax.numpy as jnp
from jax import lax
from jax.experimental import pallas as pl
from jax.experimental.pallas import tpu as pltpu
from jax.experimental.pallas import tpu_sc as plsc

B = 4096
B3 = 3 * B
D = 128
K = 64
N_REL = 64
KG_L2 = 1e-05
SOCIAL_L2 = 1e-05

NW = 32
CHUNK = 128
NCH = B3 // NW // CHUNK
ROWS_W = B3 // NW
RELS_W = B // NW

RN = 8
RM = 1024
NRB = N_REL // RN
NRW = B3 // RM


@functools.cache
def _sc_gather_fn():
  @functools.partial(
      pl.kernel,
      mesh=plsc.VectorSubcoreMesh(core_axis_name="c", subcore_axis_name="s"),
      out_type=[
          jax.ShapeDtypeStruct((B3, D), jnp.float32),
          jax.ShapeDtypeStruct((B3, D), jnp.float32),
          jax.ShapeDtypeStruct((B3, D), jnp.float32),
          jax.ShapeDtypeStruct((B, D), jnp.float32),
      ],
      scratch_types=[
          pltpu.VMEM((NCH, CHUNK), jnp.int32),
          pltpu.VMEM((NCH, CHUNK), jnp.int32),
          pltpu.VMEM((1, CHUNK), jnp.int32),
          pltpu.VMEM((ROWS_W, D), jnp.float32),
          pltpu.VMEM((ROWS_W, D), jnp.float32),
          pltpu.VMEM((RELS_W, D), jnp.float32),
          pltpu.SemaphoreType.DMA,
          pltpu.SemaphoreType.DMA,
          pltpu.SemaphoreType.DMA,
      ],
  )
  def _sc_gather(ent_idx, soc_idx, r_idx, entity_t, user_t, rel_t,
                 ent3_o, user3_o, entu3_o, relg_o,
                 idx_e, idx_s, idx_r, rows_a, rows_b, rel_rows,
                 sem_a, sem_b, sem_r):
    wid = lax.axis_index("s") * 2 + lax.axis_index("c")
    row0 = wid * ROWS_W
    rel0 = wid * RELS_W

    pltpu.sync_copy(ent_idx.at[wid], idx_e)
    pltpu.sync_copy(soc_idx.at[wid], idx_s)
    pltpu.sync_copy(r_idx.at[wid], idx_r)

    cps_a = [
        pltpu.async_copy(entity_t.at[idx_e.at[j]],
                         rows_a.at[pl.ds(j * CHUNK, CHUNK)], sem_a)
        for j in range(NCH)
    ]
    cps_b = [
        pltpu.async_copy(user_t.at[idx_s.at[j]],
                         rows_b.at[pl.ds(j * CHUNK, CHUNK)], sem_b)
        for j in range(NCH)
    ]
    cp_r = pltpu.async_copy(rel_t.at[idx_r.at[0]], rel_rows, sem_r)

    for cp in cps_a:
        cp.wait()
    pltpu.sync_copy(rows_a, ent3_o.at[pl.ds(row0, ROWS_W)])

    cps_a2 = [
        pltpu.async_copy(entity_t.at[idx_s.at[j]],
                         rows_a.at[pl.ds(j * CHUNK, CHUNK)], sem_a)
        for j in range(NCH)
    ]
    for cp in cps_b:
        cp.wait()
    pltpu.sync_copy(rows_b, user3_o.at[pl.ds(row0, ROWS_W)])

    for cp in cps_a2:
        cp.wait()
    pltpu.sync_copy(rows_a, entu3_o.at[pl.ds(row0, ROWS_W)])

    cp_r.wait()
    pltpu.sync_copy(rel_rows, relg_o.at[pl.ds(rel0, RELS_W)])

  return _sc_gather


def _norm_rows(x):
    n = jnp.sqrt(jnp.sum(x * x, axis=1, keepdims=True))
    return x / jnp.maximum(n, 1e-12)


def _l2m(x):
    return jnp.mean(jnp.sum(x * x, axis=1) / 2.0)


def _tc_body(e3_ref, tm_ref, r3_ref, relg_ref, u3_ref, eu3_ref,
             out_ref, acc_ref):
    rb = pl.program_id(0)
    rw = pl.program_id(1)

    y = jnp.dot(e3_ref[...], tm_ref[0], preferred_element_type=jnp.float32)
    rr = r3_ref[...]
    rel0 = rb * RN
    contrib = jnp.zeros((RM, K), jnp.float32)
    for j in range(RN):
        m = (rr == rel0 + j).astype(jnp.float32)
        contrib = contrib + y[:, j * K:(j + 1) * K] * m

    sl = pl.ds(rw * RM, RM)

    @pl.when(rb == 0)
    def _():
        acc_ref[sl, :] = contrib

    @pl.when(rb > 0)
    def _():
        acc_ref[sl, :] += contrib

    @pl.when(jnp.logical_and(rb == NRB - 1, rw == NRW - 1))
    def _():
        h_n = _norm_rows(acc_ref[pl.ds(0, B), :])
        p_n = _norm_rows(acc_ref[pl.ds(B, B), :])
        n_n = _norm_rows(acc_ref[pl.ds(2 * B, B), :])
        r_n = _norm_rows(relg_ref[...][:, :K])
        base = h_n + r_n
        pos_score = jnp.sum(jnp.square(base - p_n), axis=1)
        neg_score = jnp.sum(jnp.square(base - n_n), axis=1)
        x = neg_score - pos_score
        kg_loss = jnp.mean(jnp.maximum(-x, 0.0)
                           + jnp.log(1.0 + jnp.exp(-jnp.abs(x))))
        kg_l2 = _l2m(h_n) + _l2m(r_n) + _l2m(p_n) + _l2m(n_n)

        inv_u = u3_ref[pl.ds(0, B), :]
        inv_s = inv_u + eu3_ref[pl.ds(0, B), :]
        vp_s = u3_ref[pl.ds(B, B), :] + eu3_ref[pl.ds(B, B), :]
        vn_s = u3_ref[pl.ds(2 * B, B), :] + eu3_ref[pl.ds(2 * B, B), :]
        pos_s = jnp.sum(inv_s * vp_s, axis=1)
        neg_s = jnp.sum(inv_s * vn_s, axis=1)
        xs = pos_s - neg_s
        sig = 1.0 / (1.0 + jnp.exp(-xs))
        social_loss = jnp.mean(-jnp.log(1e-10 + sig))
        social_l2 = _l2m(inv_u) + _l2m(vp_s) + _l2m(vn_s)

        out_ref[0, 0] = (kg_loss + KG_L2 * kg_l2
                         + social_loss + SOCIAL_L2 * social_l2)


def _tc_call(e3, tm, r3, relg, u3, eu3):
    return pl.pallas_call(
        _tc_body,
        grid=(NRB, NRW),
        in_specs=[
            pl.BlockSpec((RM, D), lambda rb, rw: (rw, 0)),
            pl.BlockSpec((1, D, RN * K), lambda rb, rw: (rb, 0, 0)),
            pl.BlockSpec((RM, 1), lambda rb, rw: (rw, 0)),
            pl.BlockSpec((B, D), lambda rb, rw: (0, 0)),
            pl.BlockSpec((B3, D), lambda rb, rw: (0, 0)),
            pl.BlockSpec((B3, D), lambda rb, rw: (0, 0)),
        ],
        out_specs=pl.BlockSpec(memory_space=pltpu.SMEM),
        out_shape=jax.ShapeDtypeStruct((1, 1), jnp.float32),
        scratch_shapes=[pltpu.VMEM((B3, K), jnp.float32)],
        compiler_params=pltpu.CompilerParams(
            dimension_semantics=("arbitrary", "arbitrary")),
    )(e3, tm, r3, relg, u3, eu3)


def kernel(inviter_ids, voter_pos_ids, voter_neg_ids, h, r, pos_t, neg_t,
           is_train, user_embed, entity_embed, relation_embed, trans_M):
    i32 = jnp.int32
    ent_idx = jnp.concatenate([h, pos_t, neg_t]).astype(i32).reshape(
        NW, NCH, CHUNK)
    soc_idx = jnp.concatenate(
        [inviter_ids, voter_pos_ids, voter_neg_ids]).astype(i32).reshape(
        NW, NCH, CHUNK)
    r_idx = r.astype(i32).reshape(NW, 1, CHUNK)

    rel_pad = jnp.pad(relation_embed, ((0, 0), (0, D - K)))
    ent3, user3, entu3, relg = _sc_gather_fn()(
        ent_idx, soc_idx, r_idx, entity_embed, user_embed, rel_pad)

    tm = trans_M.reshape(NRB, RN, D, K).transpose(0, 2, 1, 3).reshape(
        NRB, D, RN * K)
    r3 = jnp.concatenate([r, r, r]).astype(i32).reshape(B3, 1)

    out = _tc_call(ent3, tm, r3, relg, user3, entu3)
    return out.reshape(())
